# Initial kernel scaffold; baseline (speedup 1.0000x reference)
import functools
import math

import jax
import jax.numpy as jnp
from jax import lax
from jax.experimental import pallas as pl
from jax.experimental.pallas import tpu as pltpu
from jax.experimental.pallas import tpu_sc as plsc

_N = 10000
_E = 160000
_H = 128
_R = 50
_CUT = 10.0

_NB = 1000
_EB = 800
_NC = 2
_NS = 16
_W = _NC * _NS
_EPW = _E // _W
_C = 200
_NCHUNK = _EPW // _C
_RPW = _N // _NS


def _silu(x):
    return x * jax.nn.sigmoid(x)


def _lnorm(x, g, b):
    m = jnp.mean(x, axis=-1, keepdims=True)
    v = jnp.mean((x - m) ** 2, axis=-1, keepdims=True)
    return (x - m) / jnp.sqrt(v + 1e-5) * g + b


def _dot(a, b):
    return jnp.dot(a, b, preferred_element_type=jnp.float32)


def _prep_body(s_ref, v_ref, wbot_ref, bb_ref, wv_ref, tsrc_ref, bm_ref):
    s = s_ref[...]
    tsrc_ref[:, 0:_H] = s
    bm_ref[...] = _dot(s, wbot_ref[...]) + bb_ref[...]
    wv = wv_ref[...]
    for c in range(3):
        tsrc_ref[:, _H * (c + 1):_H * (c + 2)] = _dot(v_ref[c], wv)


def _prep(S, Vflat, wbot, bb, wv):
    return pl.pallas_call(
        _prep_body,
        grid=(_N // _NB,),
        in_specs=[
            pl.BlockSpec((_NB, _H), lambda i: (i, 0)),
            pl.BlockSpec((3, _NB, _H), lambda i: (0, i, 0)),
            pl.BlockSpec((_H, _H), lambda i: (0, 0)),
            pl.BlockSpec((1, _H), lambda i: (0, 0)),
            pl.BlockSpec((_H, _H), lambda i: (0, 0)),
        ],
        out_specs=[
            pl.BlockSpec((_NB, 4 * _H), lambda i: (i, 0)),
            pl.BlockSpec((_NB, _H), lambda i: (i, 0)),
        ],
        out_shape=[
            jax.ShapeDtypeStruct((_N, 4 * _H), jnp.float32),
            jax.ShapeDtypeStruct((_N, _H), jnp.float32),
        ],
    )(S, Vflat, wbot, bb, wv)


def _gather_body(tsrc_hbm, bm_hbm, src_hbm, dst_hbm, gsrc_hbm, gdst_hbm,
                 idx_s, idx_d, rows_s, rows_d, sem):
    wid = lax.axis_index("c") * _NS + lax.axis_index("s")
    base = wid * _EPW

    def chunk(k, carry):
        off = base + k * _C
        pltpu.sync_copy(src_hbm.at[pl.ds(off, _C)], idx_s)
        pltpu.sync_copy(dst_hbm.at[pl.ds(off, _C)], idx_d)
        pltpu.async_copy(tsrc_hbm.at[idx_s], rows_s, sem).wait()
        pltpu.async_copy(bm_hbm.at[idx_d], rows_d, sem).wait()
        pltpu.sync_copy(rows_s, gsrc_hbm.at[pl.ds(off, _C)])
        pltpu.sync_copy(rows_d, gdst_hbm.at[pl.ds(off, _C)])
        return carry

    lax.fori_loop(0, _NCHUNK, chunk, 0)


def _gather(tsrc, bm, src, dst):
    f = pl.kernel(
        _gather_body,
        out_type=[
            jax.ShapeDtypeStruct((_E, 4 * _H), jnp.float32),
            jax.ShapeDtypeStruct((_E, _H), jnp.float32),
        ],
        mesh=plsc.VectorSubcoreMesh(core_axis_name="c", subcore_axis_name="s"),
        scratch_types=[
            pltpu.VMEM((_C,), jnp.int32),
            pltpu.VMEM((_C,), jnp.int32),
            pltpu.VMEM((_C, 4 * _H), jnp.float32),
            pltpu.VMEM((_C, _H), jnp.float32),
            pltpu.SemaphoreType.DMA,
        ],
    )
    return f(tsrc, bm, src, dst)


def _edge_body(gsrc_ref, gdst_ref, d_ref, ev_ref, w1, b1, w2, b2, w3, b3,
               cen, wdt, wtop, ws2, bs2, m_ref):
    d = d_ref[...]
    rbf = jnp.exp(-(((d - cen[...]) / wdt[...]) ** 2))
    cut = 0.5 * (jnp.cos(d * (math.pi / _CUT)) + 1.0)
    cut = cut * (d < _CUT).astype(jnp.float32)
    h = _silu(_dot(rbf, w1[...]) + b1[...])
    h = _silu(_dot(h, w2[...]) + b2[...])
    radial = (_dot(h, w3[...]) + b3[...]) * cut
    s_src = gsrc_ref[:, 0:_H]
    hs = _silu(_dot(s_src, wtop[...]) + gdst_ref[...])
    m_ref[0] = (_dot(hs, ws2[...]) + bs2[...]) * radial
    rs = radial * s_src
    ev = ev_ref[...]
    for c in range(3):
        m_ref[1 + c] = gsrc_ref[:, _H * (c + 1):_H * (c + 2)] * radial \
            + ev[:, c:c + 1] * rs


def _edge(gsrc, gdst, d, ev, w1, b1, w2, b2, w3, b3, cen, wdt, wtop, ws2, bs2):
    full = lambda i: (0, 0)
    return pl.pallas_call(
        _edge_body,
        grid=(_E // _EB,),
        in_specs=[
            pl.BlockSpec((_EB, 4 * _H), lambda i: (i, 0)),
            pl.BlockSpec((_EB, _H), lambda i: (i, 0)),
            pl.BlockSpec((_EB, 1), lambda i: (i, 0)),
            pl.BlockSpec((_EB, 3), lambda i: (i, 0)),
            pl.BlockSpec((_R, _H), full),
            pl.BlockSpec((1, _H), full),
            pl.BlockSpec((_H, _H), full),
            pl.BlockSpec((1, _H), full),
            pl.BlockSpec((_H, _H), full),
            pl.BlockSpec((1, _H), full),
            pl.BlockSpec((1, _R), full),
            pl.BlockSpec((1, _R), full),
            pl.BlockSpec((_H, _H), full),
            pl.BlockSpec((_H, _H), full),
            pl.BlockSpec((1, _H), full),
        ],
        out_specs=pl.BlockSpec((4, _EB, _H), lambda i: (0, i, 0)),
        out_shape=jax.ShapeDtypeStruct((4, _E, _H), jnp.float32),
    )(gsrc, gdst, d, ev, w1, b1, w2, b2, w3, b3, cen, wdt, wtop, ws2, bs2)


def _scatter_body(m_hbm, dst_hbm, zeros_hbm, p_hbm, idx_v, vals_v, acc):
    cid = lax.axis_index("c")
    sid = lax.axis_index("s")
    ebase = cid * (_E // _NC) + sid * _EPW
    rbase = sid * _RPW
    for part in range(4):
        pltpu.sync_copy(zeros_hbm.at[pl.ds(rbase, _RPW)],
                        acc.at[pl.ds(rbase, _RPW)])
        plsc.subcore_barrier()

        def chunk(k, carry):
            off = ebase + k * _C
            pltpu.sync_copy(dst_hbm.at[pl.ds(off, _C)], idx_v)
            pltpu.sync_copy(m_hbm.at[part].at[pl.ds(off, _C)], vals_v)
            pltpu.sync_copy(vals_v, acc.at[idx_v], add=True)
            return carry

        lax.fori_loop(0, _NCHUNK, chunk, 0)
        plsc.subcore_barrier()
        pltpu.sync_copy(acc.at[pl.ds(rbase, _RPW)],
                        p_hbm.at[2 * part + cid].at[pl.ds(rbase, _RPW)])
        plsc.subcore_barrier()


def _scatter(m, dst, zeros):
    f = pl.kernel(
        _scatter_body,
        out_type=jax.ShapeDtypeStruct((8, _N, _H), jnp.float32),
        mesh=plsc.VectorSubcoreMesh(core_axis_name="c", subcore_axis_name="s"),
        scratch_types=[
            pltpu.VMEM((_C,), jnp.int32),
            pltpu.VMEM((_C, _H), jnp.float32),
            pltpu.VMEM_SHARED((_N, _H), jnp.float32),
        ],
    )
    return f(m, dst, zeros)


def _node_body(s_ref, v_ref, p_ref, wsa, wsb, b1, ws2, bs2, wvu,
               sng, snb, vng, vnb, so_ref, vo_ref):
    S = s_ref[...]
    s_agg = p_ref[0] + p_ref[1]
    h = _silu(_dot(S, wsa[...]) + _dot(s_agg, wsb[...]) + b1[...])
    s_out = S + _dot(h, ws2[...]) + bs2[...]
    so_ref[...] = _lnorm(s_out, sng[...], snb[...])
    wv = wvu[...]
    for c in range(3):
        vagg = p_ref[2 + 2 * c] + p_ref[3 + 2 * c]
        vo = v_ref[c] + _dot(vagg, wv)
        vo_ref[c] = _lnorm(vo, vng[...], vnb[...])


def _node(S, Vflat, P, wsa, wsb, b1, ws2, bs2, wvu, sng, snb, vng, vnb):
    full = lambda i: (0, 0)
    return pl.pallas_call(
        _node_body,
        grid=(_N // _NB,),
        in_specs=[
            pl.BlockSpec((_NB, _H), lambda i: (i, 0)),
            pl.BlockSpec((3, _NB, _H), lambda i: (0, i, 0)),
            pl.BlockSpec((8, _NB, _H), lambda i: (0, i, 0)),
            pl.BlockSpec((_H, _H), full),
            pl.BlockSpec((_H, _H), full),
            pl.BlockSpec((1, _H), full),
            pl.BlockSpec((_H, _H), full),
            pl.BlockSpec((1, _H), full),
            pl.BlockSpec((_H, _H), full),
            pl.BlockSpec((1, _H), full),
            pl.BlockSpec((1, _H), full),
            pl.BlockSpec((1, _H), full),
            pl.BlockSpec((1, _H), full),
        ],
        out_specs=[
            pl.BlockSpec((_NB, _H), lambda i: (i, 0)),
            pl.BlockSpec((3, _NB, _H), lambda i: (0, i, 0)),
        ],
        out_shape=[
            jax.ShapeDtypeStruct((_N, _H), jnp.float32),
            jax.ShapeDtypeStruct((3, _N, _H), jnp.float32),
        ],
    )(S, Vflat, P, wsa, wsb, b1, ws2, bs2, wvu, sng, snb, vng, vnb)


def kernel(scalar_features, vector_features, edge_index, edge_vec, edge_dist,
           params):
    p = params
    S = scalar_features
    Vflat = jnp.transpose(vector_features, (2, 0, 1))
    src = edge_index[0]
    dst = edge_index[1]

    w_snet1 = p["snet1"]["W"]
    wtop, wbot = w_snet1[:_H], w_snet1[_H:]
    bb = p["snet1"]["b"].reshape(1, _H)

    tsrc, bm = _prep(S, Vflat, wbot, bb, p["vlin"]["W"])
    gsrc, gdst = _gather(tsrc, bm, src, dst)
    m = _edge(
        gsrc, gdst, edge_dist.reshape(_E, 1), edge_vec,
        p["rmlp1"]["W"], p["rmlp1"]["b"].reshape(1, _H),
        p["rmlp2"]["W"], p["rmlp2"]["b"].reshape(1, _H),
        p["rmlp3"]["W"], p["rmlp3"]["b"].reshape(1, _H),
        p["centers"].reshape(1, _R), p["widths"].reshape(1, _R),
        wtop, p["snet2"]["W"], p["snet2"]["b"].reshape(1, _H),
    )
    zeros = jnp.zeros((_N, _H), jnp.float32)
    P = _scatter(m, dst, zeros)
    s_out, v_out_f = _node(
        S, Vflat, P,
        p["supd1"]["W"][:_H], p["supd1"]["W"][_H:],
        p["supd1"]["b"].reshape(1, _H),
        p["supd2"]["W"], p["supd2"]["b"].reshape(1, _H),
        p["vupd"]["W"],
        p["sn_g"].reshape(1, _H), p["sn_b"].reshape(1, _H),
        p["vn_g"].reshape(1, _H), p["vn_b"].reshape(1, _H),
    )
    return s_out, jnp.transpose(v_out_f, (1, 2, 0))


# speedup vs baseline: 14.5982x; 14.5982x over previous
import functools
import math

import jax
import jax.numpy as jnp
from jax import lax
from jax.experimental import pallas as pl
from jax.experimental.pallas import tpu as pltpu
from jax.experimental.pallas import tpu_sc as plsc

_N = 10000
_E = 160000
_H = 128
_R = 50
_CUT = 10.0

_NB = 1000
_EB = 800
_NC = 2
_NS = 16
_W = _NC * _NS
_EPW = _E // _W
_C = 200
_NCHUNK = _EPW // _C
_NP = 10240
_RPW = _NP // _NS


def _silu(x):
    return x * jax.nn.sigmoid(x)


def _lnorm(x, g, b):
    m = jnp.mean(x, axis=-1, keepdims=True)
    v = jnp.mean((x - m) ** 2, axis=-1, keepdims=True)
    return (x - m) / jnp.sqrt(v + 1e-5) * g + b


def _dot(a, b):
    return jnp.dot(a, b, preferred_element_type=jnp.float32)


def _prep_body(s_ref, v_ref, wbot_ref, bb_ref, wv_ref, tsrc_ref, bm_ref):
    s = s_ref[...]
    tsrc_ref[:, 0:_H] = s
    bm_ref[...] = _dot(s, wbot_ref[...]) + bb_ref[...]
    wv = wv_ref[...]
    for c in range(3):
        tsrc_ref[:, _H * (c + 1):_H * (c + 2)] = _dot(v_ref[c], wv)


def _prep(S, Vflat, wbot, bb, wv):
    return pl.pallas_call(
        _prep_body,
        grid=(_N // _NB,),
        in_specs=[
            pl.BlockSpec((_NB, _H), lambda i: (i, 0)),
            pl.BlockSpec((3, _NB, _H), lambda i: (0, i, 0)),
            pl.BlockSpec((_H, _H), lambda i: (0, 0)),
            pl.BlockSpec((1, _H), lambda i: (0, 0)),
            pl.BlockSpec((_H, _H), lambda i: (0, 0)),
        ],
        out_specs=[
            pl.BlockSpec((_NB, 4 * _H), lambda i: (i, 0)),
            pl.BlockSpec((_NB, _H), lambda i: (i, 0)),
        ],
        out_shape=[
            jax.ShapeDtypeStruct((_N, 4 * _H), jnp.float32),
            jax.ShapeDtypeStruct((_N, _H), jnp.float32),
        ],
    )(S, Vflat, wbot, bb, wv)


def _gather_body(tsrc_hbm, bm_hbm, src_hbm, dst_hbm, gsrc_hbm, gdst_hbm,
                 idx_s, idx_d, rows_s, rows_d, sem):
    wid = lax.axis_index("c") * _NS + lax.axis_index("s")
    base = wid * _EPW

    def chunk(k, carry):
        off = base + k * _C
        pltpu.sync_copy(src_hbm.at[pl.ds(off, _C)], idx_s)
        pltpu.sync_copy(dst_hbm.at[pl.ds(off, _C)], idx_d)
        pltpu.async_copy(tsrc_hbm.at[idx_s], rows_s, sem).wait()
        pltpu.async_copy(bm_hbm.at[idx_d], rows_d, sem).wait()
        pltpu.sync_copy(rows_s, gsrc_hbm.at[pl.ds(off, _C)])
        pltpu.sync_copy(rows_d, gdst_hbm.at[pl.ds(off, _C)])
        return carry

    lax.fori_loop(0, _NCHUNK, chunk, 0)


def _gather(tsrc, bm, src, dst):
    f = pl.kernel(
        _gather_body,
        out_type=[
            jax.ShapeDtypeStruct((_E, 4 * _H), jnp.float32),
            jax.ShapeDtypeStruct((_E, _H), jnp.float32),
        ],
        mesh=plsc.VectorSubcoreMesh(core_axis_name="c", subcore_axis_name="s"),
        scratch_types=[
            pltpu.VMEM((_C,), jnp.int32),
            pltpu.VMEM((_C,), jnp.int32),
            pltpu.VMEM((_C, 4 * _H), jnp.float32),
            pltpu.VMEM((_C, _H), jnp.float32),
            pltpu.SemaphoreType.DMA,
        ],
    )
    return f(tsrc, bm, src, dst)


def _edge_body(gsrc_ref, gdst_ref, d_ref, ev_ref, w1, b1, w2, b2, w3, b3,
               cen, wdt, wtop, ws2, bs2, m_ref):
    d = d_ref[...]
    rbf = jnp.exp(-(((d - cen[...]) / wdt[...]) ** 2))
    cut = 0.5 * (jnp.cos(d * (math.pi / _CUT)) + 1.0)
    cut = cut * (d < _CUT).astype(jnp.float32)
    h = _silu(_dot(rbf, w1[...]) + b1[...])
    h = _silu(_dot(h, w2[...]) + b2[...])
    radial = (_dot(h, w3[...]) + b3[...]) * cut
    s_src = gsrc_ref[:, 0:_H]
    hs = _silu(_dot(s_src, wtop[...]) + gdst_ref[...])
    m_ref[0] = (_dot(hs, ws2[...]) + bs2[...]) * radial
    rs = radial * s_src
    ev = ev_ref[...]
    for c in range(3):
        m_ref[1 + c] = gsrc_ref[:, _H * (c + 1):_H * (c + 2)] * radial \
            + ev[:, c:c + 1] * rs


def _edge(gsrc, gdst, d, ev, w1, b1, w2, b2, w3, b3, cen, wdt, wtop, ws2, bs2):
    full = lambda i: (0, 0)
    return pl.pallas_call(
        _edge_body,
        grid=(_E // _EB,),
        in_specs=[
            pl.BlockSpec((_EB, 4 * _H), lambda i: (i, 0)),
            pl.BlockSpec((_EB, _H), lambda i: (i, 0)),
            pl.BlockSpec((_EB, 1), lambda i: (i, 0)),
            pl.BlockSpec((_EB, 3), lambda i: (i, 0)),
            pl.BlockSpec((_R, _H), full),
            pl.BlockSpec((1, _H), full),
            pl.BlockSpec((_H, _H), full),
            pl.BlockSpec((1, _H), full),
            pl.BlockSpec((_H, _H), full),
            pl.BlockSpec((1, _H), full),
            pl.BlockSpec((1, _R), full),
            pl.BlockSpec((1, _R), full),
            pl.BlockSpec((_H, _H), full),
            pl.BlockSpec((_H, _H), full),
            pl.BlockSpec((1, _H), full),
        ],
        out_specs=pl.BlockSpec((4, _EB, _H), lambda i: (0, i, 0)),
        out_shape=jax.ShapeDtypeStruct((4, _E, _H), jnp.float32),
    )(gsrc, gdst, d, ev, w1, b1, w2, b2, w3, b3, cen, wdt, wtop, ws2, bs2)


def _scatter_body(m_hbm, dst_hbm, zeros_hbm, p_hbm, idx_v, vals_v, acc):
    cid = lax.axis_index("c")
    sid = lax.axis_index("s")
    ebase = cid * (_E // _NC) + sid * _EPW
    rbase = sid * _RPW
    for part in range(4):
        pltpu.sync_copy(zeros_hbm.at[pl.ds(rbase, _RPW)],
                        acc.at[pl.ds(rbase, _RPW)])
        plsc.subcore_barrier()

        def chunk(k, carry):
            off = ebase + k * _C
            pltpu.sync_copy(dst_hbm.at[pl.ds(off, _C)], idx_v)
            pltpu.sync_copy(m_hbm.at[part].at[pl.ds(off, _C)], vals_v)
            pltpu.sync_copy(vals_v, acc.at[idx_v], add=True)
            return carry

        lax.fori_loop(0, _NCHUNK, chunk, 0)
        plsc.subcore_barrier()
        pltpu.sync_copy(acc.at[pl.ds(rbase, _RPW)],
                        p_hbm.at[2 * part + cid].at[pl.ds(rbase, _RPW)])
        plsc.subcore_barrier()


def _scatter(m, dst, zeros):
    f = pl.kernel(
        _scatter_body,
        out_type=jax.ShapeDtypeStruct((8, _NP, _H), jnp.float32),
        mesh=plsc.VectorSubcoreMesh(core_axis_name="c", subcore_axis_name="s"),
        scratch_types=[
            pltpu.VMEM((_C,), jnp.int32),
            pltpu.VMEM((_C, _H), jnp.float32),
            pltpu.VMEM_SHARED((_NP, _H), jnp.float32),
        ],
    )
    return f(m, dst, zeros)


def _node_body(s_ref, v_ref, p_ref, wsa, wsb, b1, ws2, bs2, wvu,
               sng, snb, vng, vnb, so_ref, vo_ref):
    S = s_ref[...]
    s_agg = p_ref[0] + p_ref[1]
    h = _silu(_dot(S, wsa[...]) + _dot(s_agg, wsb[...]) + b1[...])
    s_out = S + _dot(h, ws2[...]) + bs2[...]
    so_ref[...] = _lnorm(s_out, sng[...], snb[...])
    wv = wvu[...]
    for c in range(3):
        vagg = p_ref[2 + 2 * c] + p_ref[3 + 2 * c]
        vo = v_ref[c] + _dot(vagg, wv)
        vo_ref[c] = _lnorm(vo, vng[...], vnb[...])


def _node(S, Vflat, P, wsa, wsb, b1, ws2, bs2, wvu, sng, snb, vng, vnb):
    full = lambda i: (0, 0)
    return pl.pallas_call(
        _node_body,
        grid=(_N // _NB,),
        in_specs=[
            pl.BlockSpec((_NB, _H), lambda i: (i, 0)),
            pl.BlockSpec((3, _NB, _H), lambda i: (0, i, 0)),
            pl.BlockSpec((8, _NB, _H), lambda i: (0, i, 0)),
            pl.BlockSpec((_H, _H), full),
            pl.BlockSpec((_H, _H), full),
            pl.BlockSpec((1, _H), full),
            pl.BlockSpec((_H, _H), full),
            pl.BlockSpec((1, _H), full),
            pl.BlockSpec((_H, _H), full),
            pl.BlockSpec((1, _H), full),
            pl.BlockSpec((1, _H), full),
            pl.BlockSpec((1, _H), full),
            pl.BlockSpec((1, _H), full),
        ],
        out_specs=[
            pl.BlockSpec((_NB, _H), lambda i: (i, 0)),
            pl.BlockSpec((3, _NB, _H), lambda i: (0, i, 0)),
        ],
        out_shape=[
            jax.ShapeDtypeStruct((_N, _H), jnp.float32),
            jax.ShapeDtypeStruct((3, _N, _H), jnp.float32),
        ],
    )(S, Vflat, P, wsa, wsb, b1, ws2, bs2, wvu, sng, snb, vng, vnb)


def kernel(scalar_features, vector_features, edge_index, edge_vec, edge_dist,
           params):
    p = params
    S = scalar_features
    Vflat = jnp.transpose(vector_features, (2, 0, 1))
    src = edge_index[0]
    dst = edge_index[1]

    w_snet1 = p["snet1"]["W"]
    wtop, wbot = w_snet1[:_H], w_snet1[_H:]
    bb = p["snet1"]["b"].reshape(1, _H)

    tsrc, bm = _prep(S, Vflat, wbot, bb, p["vlin"]["W"])
    gsrc, gdst = _gather(tsrc, bm, src, dst)
    m = _edge(
        gsrc, gdst, edge_dist.reshape(_E, 1), edge_vec,
        p["rmlp1"]["W"], p["rmlp1"]["b"].reshape(1, _H),
        p["rmlp2"]["W"], p["rmlp2"]["b"].reshape(1, _H),
        p["rmlp3"]["W"], p["rmlp3"]["b"].reshape(1, _H),
        p["centers"].reshape(1, _R), p["widths"].reshape(1, _R),
        wtop, p["snet2"]["W"], p["snet2"]["b"].reshape(1, _H),
    )
    zeros = jnp.zeros((_NP, _H), jnp.float32)
    P = _scatter(m, dst, zeros)
    s_out, v_out_f = _node(
        S, Vflat, P,
        p["supd1"]["W"][:_H], p["supd1"]["W"][_H:],
        p["supd1"]["b"].reshape(1, _H),
        p["supd2"]["W"], p["supd2"]["b"].reshape(1, _H),
        p["vupd"]["W"],
        p["sn_g"].reshape(1, _H), p["sn_b"].reshape(1, _H),
        p["vn_g"].reshape(1, _H), p["vn_b"].reshape(1, _H),
    )
    return s_out, jnp.transpose(v_out_f, (1, 2, 0))
